# baseline (speedup 1.0000x reference)
import functools

import jax
import jax.numpy as jnp
from jax import lax
from jax.experimental import pallas as pl
from jax.experimental.pallas import tpu as pltpu
from jax.experimental.pallas import tpu_sc as plsc

_NW = 32
_CHUNK = 128
_NBUF = 5


def _build_pair_gather(nchunk, vp):
  mesh = plsc.VectorSubcoreMesh(core_axis_name="c", subcore_axis_name="s")
  b_per_w = nchunk * _CHUNK

  @functools.partial(
      pl.kernel,
      out_type=jax.ShapeDtypeStruct((_NW * b_per_w, 128), jnp.float32),
      mesh=mesh,
      scratch_types=[
          pltpu.VMEM((nchunk, _CHUNK), jnp.int32),
      ] + [pltpu.VMEM((_CHUNK, 128), jnp.float32) for _ in range(_NBUF)]
        + [pltpu.SemaphoreType.DMA for _ in range(_NBUF)],
      compiler_params=pltpu.CompilerParams(use_tc_tiling_on_sc=True),
  )
  def k(idx_hbm, table_hbm, out_hbm, idx_v, *scratch):
    bufs = scratch[:_NBUF]
    sems = scratch[_NBUF:]
    wid = lax.axis_index("s") * 2 + lax.axis_index("c")
    base = wid * b_per_w
    pltpu.sync_copy(idx_hbm.at[wid], idx_v)
    for b in range(_NBUF):
      pltpu.async_copy(table_hbm.at[idx_v.at[b]], bufs[b], sems[b])

    def body(g, carry):
      for b in range(_NBUF):
        j = g * _NBUF + b
        pltpu.make_async_copy(table_hbm.at[idx_v.at[j]], bufs[b], sems[b]).wait()
        pltpu.sync_copy(bufs[b], out_hbm.at[pl.ds(base + j * _CHUNK, _CHUNK)])
        pltpu.async_copy(table_hbm.at[idx_v.at[j + _NBUF]], bufs[b], sems[b])
      return carry

    steady = (nchunk - _NBUF) // _NBUF
    lax.fori_loop(0, steady, body, 0)
    for b in range(_NBUF):
      j = steady * _NBUF + b
      pltpu.make_async_copy(table_hbm.at[idx_v.at[j]], bufs[b], sems[b]).wait()
      pltpu.sync_copy(bufs[b], out_hbm.at[pl.ds(base + j * _CHUNK, _CHUNK)])

  return k


def _select_body(pairs_ref, m_ref, o_ref):
  lo = pairs_ref[:, 0:64]
  hi = pairs_ref[:, 64:128]
  m = m_ref[...]
  o_ref[...] = lo + (hi - lo) * m


def _pair_select(pairs, mask, total, d):
  rows = 800
  grid = total // rows
  return pl.pallas_call(
      _select_body,
      grid=(grid,),
      in_specs=[
          pl.BlockSpec((rows, 2 * d), lambda g: (g, 0)),
          pl.BlockSpec((rows, d), lambda g: (g, 0)),
      ],
      out_specs=pl.BlockSpec((rows, d), lambda g: (g, 0)),
      out_shape=jax.ShapeDtypeStruct((total, d), jnp.float32),
  )(pairs, mask)


def kernel(inputs, embeddings):
  bsz, seq = inputs.shape
  v, d = embeddings.shape
  total = bsz * seq
  nchunk = total // (_NW * _CHUNK)
  idx = inputs.astype(jnp.int32).reshape(total)
  idx_hi3 = (idx >> 1).reshape(_NW, nchunk, _CHUNK)
  mask = jnp.broadcast_to((idx & 1).astype(jnp.float32)[:, None], (total, d))
  table2 = embeddings.reshape(v // 2, 2 * d)
  pairs = _build_pair_gather(nchunk, v // 2)(idx_hi3, table2)
  out2 = _pair_select(pairs, mask, total, d)
  return out2.reshape(bsz, seq, d)


# speedup vs baseline: 1.7195x; 1.7195x over previous
import functools

import jax
import jax.numpy as jnp
from jax import lax
from jax.experimental import pallas as pl
from jax.experimental.pallas import tpu as pltpu
from jax.experimental.pallas import tpu_sc as plsc

_NW = 32


def _build_gather(bsz, seq, d):
  mesh = plsc.VectorSubcoreMesh(core_axis_name="c", subcore_axis_name="s")
  b_per_w = bsz // _NW

  @functools.partial(
      pl.kernel,
      out_type=jax.ShapeDtypeStruct((bsz, seq, d), jnp.float32),
      mesh=mesh,
      scratch_types=[
          pltpu.VMEM((seq,), jnp.int32),
          pltpu.VMEM((seq, d), jnp.float32),
          pltpu.SemaphoreType.DMA,
      ],
      compiler_params=pltpu.CompilerParams(use_tc_tiling_on_sc=True),
  )
  def k(idx_hbm, table_hbm, out_hbm, idx_v, vbuf, sem):
    wid = lax.axis_index("s") * 2 + lax.axis_index("c")

    def body(t, carry):
      b = wid * b_per_w + t
      pltpu.sync_copy(idx_hbm.at[b], idx_v)
      for g in range(seq // 16):
        v = idx_v[pl.ds(16 * g, 16)]
        for l in range(16):
          pltpu.async_copy(table_hbm.at[v[l]], vbuf.at[16 * g + l], sem)
      rem = seq % 16
      if rem:
        v = idx_v[pl.ds(seq - 16, 16)]
        for l in range(16 - rem, 16):
          pltpu.async_copy(table_hbm.at[v[l]], vbuf.at[seq - 16 + l], sem)
      for r in range(seq):
        pltpu.make_async_copy(table_hbm.at[0], vbuf.at[r], sem).wait()
      pltpu.sync_copy(vbuf, out_hbm.at[b])
      return carry

    lax.fori_loop(0, b_per_w, body, 0)

  return k


def kernel(inputs, embeddings):
  bsz, seq = inputs.shape
  v, d = embeddings.shape
  idx = inputs.astype(jnp.int32)
  return _build_gather(bsz, seq, d)(idx, embeddings)


# speedup vs baseline: 2.1449x; 1.2474x over previous
import functools

import jax
import jax.numpy as jnp
from jax import lax
from jax.experimental import pallas as pl
from jax.experimental.pallas import tpu as pltpu
from jax.experimental.pallas import tpu_sc as plsc

_NW = 32


def _build_gather(bsz, seq, d):
  mesh = plsc.VectorSubcoreMesh(core_axis_name="c", subcore_axis_name="s")
  b_per_w = bsz // _NW

  @functools.partial(
      pl.kernel,
      out_type=jax.ShapeDtypeStruct((bsz, seq, d), jnp.float32),
      mesh=mesh,
      scratch_types=[
          pltpu.VMEM((seq,), jnp.int32),
          pltpu.VMEM((seq,), jnp.int32),
          pltpu.VMEM((seq, d), jnp.float32),
          pltpu.VMEM((seq, d), jnp.float32),
      ] + [pltpu.SemaphoreType.DMA for _ in range(6)],
      compiler_params=pltpu.CompilerParams(use_tc_tiling_on_sc=True),
  )
  def k(idx_hbm, table_hbm, out_hbm, iv0, iv1, vb0, vb1, *sems):
    iv = (iv0, iv1)
    vb = (vb0, vb1)
    si = sems[0:2]
    sr = sems[2:4]
    so = sems[4:6]
    wid = lax.axis_index("s") * 2 + lax.axis_index("c")
    b0 = wid * b_per_w

    def issue(p, b, first):
      pltpu.make_async_copy(idx_hbm.at[b], iv[p], si[p]).wait()
      if not first:
        pltpu.make_async_copy(vb[p], out_hbm.at[b], so[p]).wait()
      for g in range(seq // 16):
        v = iv[p][pl.ds(16 * g, 16)]
        for l in range(16):
          pltpu.async_copy(table_hbm.at[v[l]], vb[p].at[16 * g + l], sr[p])
      rem = seq % 16
      if rem:
        v = iv[p][pl.ds(seq - 16, 16)]
        for l in range(16 - rem, 16):
          pltpu.async_copy(table_hbm.at[v[l]], vb[p].at[seq - 16 + l], sr[p])
      pltpu.async_copy(idx_hbm.at[jnp.minimum(b + 2, bsz - 1)], iv[p], si[p])

    def finish(p, b):
      pltpu.make_async_copy(out_hbm.at[b], vb[p], sr[p]).wait()
      pltpu.async_copy(vb[p], out_hbm.at[b], so[p])

    pltpu.async_copy(idx_hbm.at[b0], iv[0], si[0])
    pltpu.async_copy(idx_hbm.at[b0 + 1], iv[1], si[1])
    issue(0, b0, True)
    issue(1, b0 + 1, True)

    def body(h, carry):
      b = b0 + 2 * h
      finish(0, b - 2)
      issue(0, b, False)
      finish(1, b - 1)
      issue(1, b + 1, False)
      return carry

    lax.fori_loop(1, b_per_w // 2, body, 0)
    finish(0, b0 + b_per_w - 2)
    finish(1, b0 + b_per_w - 1)
    pltpu.make_async_copy(idx_hbm.at[b0], iv[0], si[0]).wait()
    pltpu.make_async_copy(idx_hbm.at[b0], iv[1], si[1]).wait()
    pltpu.make_async_copy(vb[0], out_hbm.at[b0], so[0]).wait()
    pltpu.make_async_copy(vb[1], out_hbm.at[b0], so[1]).wait()

  return k


def kernel(inputs, embeddings):
  bsz, seq = inputs.shape
  v, d = embeddings.shape
  idx = inputs.astype(jnp.int32)
  return _build_gather(bsz, seq, d)(idx, embeddings)


# speedup vs baseline: 2.2408x; 1.0447x over previous
import functools

import jax
import jax.numpy as jnp
from jax import lax
from jax.experimental import pallas as pl
from jax.experimental.pallas import tpu as pltpu
from jax.experimental.pallas import tpu_sc as plsc

_NW = 32
_DEPTH = 4


def _build_gather(bsz, seq, d):
  mesh = plsc.VectorSubcoreMesh(core_axis_name="c", subcore_axis_name="s")
  b_per_w = bsz // _NW

  @functools.partial(
      pl.kernel,
      out_type=jax.ShapeDtypeStruct((bsz, seq, d), jnp.float32),
      mesh=mesh,
      scratch_types=[pltpu.VMEM((seq,), jnp.int32) for _ in range(_DEPTH)]
      + [pltpu.VMEM((seq, d), jnp.float32) for _ in range(_DEPTH)]
      + [pltpu.SemaphoreType.DMA for _ in range(3 * _DEPTH)],
      compiler_params=pltpu.CompilerParams(use_tc_tiling_on_sc=True),
  )
  def k(idx_hbm, table_hbm, out_hbm, *refs):
    iv = refs[0:_DEPTH]
    vb = refs[_DEPTH:2 * _DEPTH]
    si = refs[2 * _DEPTH:3 * _DEPTH]
    sr = refs[3 * _DEPTH:4 * _DEPTH]
    so = refs[4 * _DEPTH:5 * _DEPTH]
    wid = lax.axis_index("s") * 2 + lax.axis_index("c")
    b0 = wid * b_per_w

    def issue(p, b, first):
      pltpu.make_async_copy(idx_hbm.at[b], iv[p], si[p]).wait()
      if not first:
        pltpu.make_async_copy(vb[p], out_hbm.at[b], so[p]).wait()
      for g in range(seq // 16):
        v = iv[p][pl.ds(16 * g, 16)]
        for l in range(16):
          pltpu.async_copy(table_hbm.at[v[l]], vb[p].at[16 * g + l], sr[p])
      rem = seq % 16
      if rem:
        v = iv[p][pl.ds(seq - 16, 16)]
        for l in range(16 - rem, 16):
          pltpu.async_copy(table_hbm.at[v[l]], vb[p].at[seq - 16 + l], sr[p])
      pltpu.async_copy(idx_hbm.at[jnp.minimum(b + _DEPTH, bsz - 1)], iv[p], si[p])

    def finish(p, b):
      pltpu.make_async_copy(out_hbm.at[b], vb[p], sr[p]).wait()
      pltpu.async_copy(vb[p], out_hbm.at[b], so[p])

    for p in range(_DEPTH):
      pltpu.async_copy(idx_hbm.at[b0 + p], iv[p], si[p])
    for p in range(_DEPTH):
      issue(p, b0 + p, True)

    def body(h, carry):
      b = b0 + _DEPTH * h
      for p in range(_DEPTH):
        finish(p, b - _DEPTH + p)
        issue(p, b + p, False)
      return carry

    lax.fori_loop(1, b_per_w // _DEPTH, body, 0)
    for p in range(_DEPTH):
      finish(p, b0 + b_per_w - _DEPTH + p)
    for p in range(_DEPTH):
      pltpu.make_async_copy(idx_hbm.at[b0], iv[p], si[p]).wait()
      pltpu.make_async_copy(vb[p], out_hbm.at[b0], so[p]).wait()

  return k


def kernel(inputs, embeddings):
  bsz, seq = inputs.shape
  v, d = embeddings.shape
  idx = inputs.astype(jnp.int32)
  return _build_gather(bsz, seq, d)(idx, embeddings)
